# baseline (speedup 1.0000x reference)
import jax
import jax.numpy as jnp
from jax.experimental import pallas as pl
from jax.experimental.pallas import tpu as pltpu

_DIM = 768
_NUM_OUT = 64
_BM = 2048
_S = 2

_NEG_INF = float("-inf")


def _top2_softmax_t(logits):
    iota = jax.lax.broadcasted_iota(jnp.int32, logits.shape, 0).astype(jnp.float32)
    big = float(_NUM_OUT)

    v1 = jnp.max(logits, axis=0, keepdims=True)
    i1f = jnp.min(jnp.where(logits == v1, iota, big), axis=0, keepdims=True)
    masked = jnp.where(iota == i1f, _NEG_INF, logits)
    v2 = jnp.max(masked, axis=0, keepdims=True)
    i2f = jnp.min(jnp.where(masked == v2, iota, big), axis=0, keepdims=True)

    t = jnp.exp(v2 - v1)
    denom = 1.0 + t
    probs = jnp.concatenate([1.0 / denom, t / denom], axis=0)
    idx = jnp.concatenate([i1f.astype(jnp.int32), i2f.astype(jnp.int32)], axis=0)
    return probs, idx


def _router_block(*refs):
    x_refs = refs[:_S]
    w_ref = refs[_S]
    b_ref = refs[_S + 1]
    out_refs = refs[_S + 2:]
    w = w_ref[...]
    bias = b_ref[...]
    for s in range(_S):
        x = x_refs[s][...]
        logits = jax.lax.dot_general(
            w, x, (((1,), (1,)), ((), ())), preferred_element_type=jnp.float32
        )
        probs, idx = _top2_softmax_t(logits + bias)
        out_refs[2 * s][...] = probs
        out_refs[2 * s + 1][...] = idx


def kernel(input, W, b):
    n_tok = input.shape[0]
    chunk = n_tok // _S
    steps = chunk // _BM
    b2d = b.reshape(_NUM_OUT, 1)

    in_specs = [
        pl.BlockSpec((_BM, _DIM), lambda i, s=s: (s * steps + i, 0))
        for s in range(_S)
    ]
    in_specs += [
        pl.BlockSpec((_NUM_OUT, _DIM), lambda i: (0, 0)),
        pl.BlockSpec((_NUM_OUT, 1), lambda i: (0, 0)),
    ]
    out_specs = []
    out_shape = []
    for s in range(_S):
        out_specs += [
            pl.BlockSpec((2, _BM), lambda i: (0, i)),
            pl.BlockSpec((2, _BM), lambda i: (0, i)),
        ]
        out_shape += [
            jax.ShapeDtypeStruct((2, chunk), jnp.float32),
            jax.ShapeDtypeStruct((2, chunk), jnp.int32),
        ]

    outs = pl.pallas_call(
        _router_block,
        grid=(steps,),
        in_specs=in_specs,
        out_specs=out_specs,
        out_shape=out_shape,
        compiler_params=pltpu.CompilerParams(
            dimension_semantics=("arbitrary",),
        ),
    )(*([input] * _S), W, b2d)

    probs_t = jnp.concatenate(outs[0::2], axis=1)
    idx_t = jnp.concatenate(outs[1::2], axis=1)
    return probs_t.T, idx_t.T


# speedup vs baseline: 1.0989x; 1.0989x over previous
import jax
import jax.numpy as jnp
from jax.experimental import pallas as pl
from jax.experimental.pallas import tpu as pltpu

_DIM = 768
_NUM_OUT = 64
_BM = 4096

_NEG_INF = float("-inf")


def _router_block(x_ref, w_ref, b_ref, probs_ref, idx_ref):
    x = x_ref[...]
    w = w_ref[...]
    logits = jax.lax.dot_general(
        w, x, (((1,), (1,)), ((), ())), preferred_element_type=jnp.float32
    )
    logits = logits + b_ref[...]

    iota = jax.lax.broadcasted_iota(jnp.int32, logits.shape, 0).astype(jnp.float32)
    big = float(_NUM_OUT)

    v1 = jnp.max(logits, axis=0, keepdims=True)
    i1f = jnp.min(jnp.where(logits == v1, iota, big), axis=0, keepdims=True)
    masked = jnp.where(iota == i1f, _NEG_INF, logits)
    v2 = jnp.max(masked, axis=0, keepdims=True)
    i2f = jnp.min(jnp.where(masked == v2, iota, big), axis=0, keepdims=True)

    t = jnp.exp(v2 - v1)
    denom = 1.0 + t
    probs_ref[...] = jnp.concatenate([1.0 / denom, t / denom], axis=0)
    idx_ref[...] = jnp.concatenate(
        [i1f.astype(jnp.int32), i2f.astype(jnp.int32)], axis=0
    )


def kernel(input, W, b):
    n_tok = input.shape[0]
    grid = (n_tok // _BM,)
    b2d = b.reshape(_NUM_OUT, 1)
    probs_t, idx_t = pl.pallas_call(
        _router_block,
        grid=grid,
        in_specs=[
            pl.BlockSpec((_BM, _DIM), lambda i: (i, 0)),
            pl.BlockSpec((_NUM_OUT, _DIM), lambda i: (0, 0)),
            pl.BlockSpec((_NUM_OUT, 1), lambda i: (0, 0)),
        ],
        out_specs=[
            pl.BlockSpec((2, _BM), lambda i: (0, i)),
            pl.BlockSpec((2, _BM), lambda i: (0, i)),
        ],
        out_shape=[
            jax.ShapeDtypeStruct((2, n_tok), jnp.float32),
            jax.ShapeDtypeStruct((2, n_tok), jnp.int32),
        ],
        compiler_params=pltpu.CompilerParams(
            dimension_semantics=("parallel",),
        ),
    )(input, W, b2d)
    return probs_t.T, idx_t.T
